# baseline (speedup 1.0000x reference)
import jax
import jax.numpy as jnp
from jax import lax
from jax.experimental import pallas as pl
from jax.experimental.pallas import tpu as pltpu
from jax.experimental.pallas import tpu_sc as plsc

N = 10000
E = 640000
D = 128
NEG = -1e30

NW = 32
SEG = 2 * N
SPT = SEG // NW
CH = 2000
NCHUNK = E // CH
FB = 512
RING = 2 * FB
GB = 32
NBUF = 8
STRIDE = E + 2 * FB
KMASK = 32767


def _leaky(x):
    return jnp.where(x >= 0, x, 0.01 * x)


def _pre_body(desc_ref, tw_ref, nm_ref, ct_ref, Wd, bd, Wt, bt, Wn, bn,
              Wc, bc, Win, bin_, x_ref):
    a = _leaky(jnp.dot(desc_ref[...], Wd[...],
                       preferred_element_type=jnp.float32) + bd[...])
    b = _leaky(jnp.dot(tw_ref[...], Wt[...],
                       preferred_element_type=jnp.float32) + bt[...])
    c = _leaky(jnp.dot(nm_ref[...], Wn[...],
                       preferred_element_type=jnp.float32) + bn[...])
    d = _leaky(jnp.dot(ct_ref[...], Wc[...],
                       preferred_element_type=jnp.float32) + bc[...])
    h = jnp.concatenate([a, b, c, d], axis=1)
    x_ref[...] = _leaky(jnp.dot(h, Win[...],
                                preferred_element_type=jnp.float32) + bin_[...])


def _pre(desc, tw, nm, ct, Wd, bd, Wt, bt, Wn, bn, Wc, bc, Win, bin_):
    BR = 1000
    grid = (N // BR,)
    row_bs = lambda cols: pl.BlockSpec((BR, cols), lambda i: (i, 0))
    full = lambda s: pl.BlockSpec(s, lambda i: (0,) * len(s))
    return pl.pallas_call(
        _pre_body,
        grid=grid,
        in_specs=[row_bs(768), row_bs(768), row_bs(5), row_bs(3),
                  full((768, 32)), full((1, 32)), full((768, 32)), full((1, 32)),
                  full((5, 32)), full((1, 32)), full((3, 32)), full((1, 32)),
                  full((D, D)), full((1, D))],
        out_specs=row_bs(D),
        out_shape=jax.ShapeDtypeStruct((N, D), jnp.float32),
    )(desc, tw, nm, ct, Wd, bd, Wt, bt, Wn, bn, Wc, bc, Win, bin_)


def _pack_body(s_ref, d_ref, t_ref, pk_ref):
    pk_ref[...] = (s_ref[...] << 15) | (d_ref[...] + t_ref[...] * N)


def _pack(src, dst, typ):
    shp = (E // 128, 128)
    return pl.pallas_call(
        _pack_body,
        out_shape=jax.ShapeDtypeStruct(shp, jnp.int32),
    )(src.reshape(shp), dst.reshape(shp), typ.reshape(shp)).reshape(E)


def _route_body(pk_hbm, slist_hbm, llist_hbm, counts_hbm,
                pkv, sring, lring, cntv, c0, c1):
    csems = (c0, c1)
    wid = lax.axis_index("s") * 2 + lax.axis_index("c")
    lo = wid * SPT
    lbase = wid * STRIDE

    def init_ring(i, _):
        sring[pl.ds(i * 16, 16)] = jnp.zeros((16,), jnp.int32)
        lring[pl.ds(i * 16, 16)] = jnp.full((16,), SPT, jnp.int32)
        return 0
    lax.fori_loop(0, RING // 16, init_ring, 0)

    def chunk_slices(ci, p):
        hoff = pl.multiple_of(ci * CH, 16)
        return pk_hbm.at[pl.ds(hoff, CH)], pkv.at[pl.ds(p * CH, CH)]

    def issue(ci, p):
        s, v = chunk_slices(ci, p)
        pltpu.async_copy(s, v, csems[p])

    def wait_chunk(p):
        s, v = chunk_slices(0, p)
        pltpu.make_async_copy(s, v, csems[p]).wait()

    def flush(block, sel):
        boff = pl.multiple_of(sel * FB, FB)
        hoff = pl.multiple_of(lbase + block * FB, FB)
        pltpu.sync_copy(sring.at[pl.ds(boff, FB)], slist_hbm.at[pl.ds(hoff, FB)])
        pltpu.sync_copy(lring.at[pl.ds(boff, FB)], llist_hbm.at[pl.ds(hoff, FB)])

    def scan_one(off, c):
        cnt, flushed = c
        pkx = pkv[pl.ds(off, 16)]
        rel = (pkx & KMASK) - lo
        msk = plsc.bitcast(rel, jnp.uint32) < jnp.uint32(SPT)
        csum = jnp.cumsum(jnp.where(msk, 1, 0).astype(jnp.int32))
        idx = (cnt + csum - 1) & (RING - 1)
        plsc.store_scatter(sring, [idx], pkx >> 15, mask=msk)
        plsc.store_scatter(lring, [idx], rel, mask=msk)
        return cnt + plsc.all_reduce_population_count(msk)[0], flushed

    def scan_slot(p, carry):
        def scan_v(v, c):
            base = p * CH + v * 16
            cnt, flushed = scan_one(base, c)

            @pl.when(cnt - flushed >= FB)
            def _():
                flush(flushed // FB, (flushed // FB) & 1)
            flushed = jnp.where(cnt - flushed >= FB, flushed + FB, flushed)
            return (cnt, flushed)
        return lax.fori_loop(0, CH // 16, scan_v, carry)

    issue(0, 0)
    issue(1, 1)

    def super_step(c2, carry):
        for p in range(2):
            wait_chunk(p)
            carry = scan_slot(p, carry)
            issue(c2 * 2 + p + 2, p)
        return carry

    carry = lax.fori_loop(0, NCHUNK // 2 - 1, super_step,
                          (jnp.int32(0), jnp.int32(0)))
    for p in range(2):
        wait_chunk(p)
        carry = scan_slot(p, carry)
    cnt, flushed = carry

    pad_idx = (cnt + lax.iota(jnp.int32, 16)) & (RING - 1)
    plsc.store_scatter(sring, [pad_idx], jnp.zeros((16,), jnp.int32))
    plsc.store_scatter(lring, [pad_idx], jnp.full((16,), SPT, jnp.int32))
    nblk = (cnt + 16 + FB - 1) // FB
    for extra in range(2):
        bidx = flushed // FB + extra

        @pl.when(bidx < nblk)
        def _():
            flush(bidx, bidx & 1)

    cntv[pl.ds(0, 16)] = jnp.broadcast_to(nblk * FB, (16,)).astype(jnp.int32)
    pltpu.sync_copy(cntv, counts_hbm.at[pl.ds(pl.multiple_of(wid * 16, 16), 16)])


def _route(pk):
    mesh = plsc.VectorSubcoreMesh(core_axis_name="c", subcore_axis_name="s")
    f = pl.kernel(
        _route_body,
        out_type=(jax.ShapeDtypeStruct((NW * STRIDE,), jnp.int32),
                  jax.ShapeDtypeStruct((NW * STRIDE,), jnp.int32),
                  jax.ShapeDtypeStruct((NW * 16,), jnp.int32)),
        mesh=mesh,
        compiler_params=pltpu.CompilerParams(needs_layout_passes=False),
        scratch_types=[
            pltpu.VMEM((2 * CH,), jnp.int32),
            pltpu.VMEM((RING,), jnp.int32),
            pltpu.VMEM((RING,), jnp.int32),
            pltpu.VMEM((16,), jnp.int32),
            pltpu.SemaphoreType.DMA,
            pltpu.SemaphoreType.DMA,
        ],
    )
    return f(pk)


def _drain_body(x_hbm, slist_hbm, llist_hbm, counts_hbm, out_hbm,
                idxb, locb, cntv, rows, acc, s0, s1, s2, s3, s4, s5, s6, s7):
    sems = (s0, s1, s2, s3, s4, s5, s6, s7)
    wid = lax.axis_index("s") * 2 + lax.axis_index("c")
    lo = wid * SPT
    lbase = wid * STRIDE

    def init_row(i, _):
        acc[pl.ds(i * 16, 16)] = jnp.full((16,), NEG, jnp.float32)
        return 0
    lax.fori_loop(0, (SPT + 1) * D // 16, init_row, 0)

    pltpu.sync_copy(counts_hbm.at[pl.ds(pl.multiple_of(wid * 16, 16), 16)], cntv)
    m = cntv[pl.ds(0, 16)][0]
    nb = m // GB

    def load_issue(b, p):
        hoff = pl.multiple_of(lbase + b * GB, GB)
        islc = idxb.at[pl.ds(p * GB, GB)]
        pltpu.sync_copy(slist_hbm.at[pl.ds(hoff, GB)], islc)
        pltpu.sync_copy(llist_hbm.at[pl.ds(hoff, GB)],
                        locb.at[pl.ds(p * GB, GB)])
        pltpu.async_copy(x_hbm.at[islc], rows.at[pl.ds(p * GB, GB)], sems[p])

    def wait_slot(p):
        pltpu.make_async_copy(x_hbm.at[idxb.at[pl.ds(p * GB, GB)]],
                              rows.at[pl.ds(p * GB, GB)], sems[p]).wait()

    def rmw_slot(p):
        def group_step(q, _):
            locs = locb[pl.ds(p * GB + q * 16, 16)]
            for k16 in range(16):
                rbase = pl.multiple_of(locs[k16] * D, D)
                r = p * GB + q * 16 + k16
                for k in range(8):
                    sl = pl.ds(rbase + k * 16, 16)
                    acc[sl] = jnp.maximum(acc[sl], rows[r, pl.ds(k * 16, 16)])
            return 0
        lax.fori_loop(0, GB // 16, group_step, 0)

    for p in range(NBUF):
        load_issue(jnp.int32(p), p)

    def super_step(sb, _):
        for p in range(NBUF):
            wait_slot(p)
            rmw_slot(p)
            load_issue((sb + 1) * NBUF + p, p)
        return 0
    lax.fori_loop(0, nb // NBUF - 1, super_step, 0)

    for p in range(NBUF):
        wait_slot(p)
        rmw_slot(p)

    def fix_row(i, _):
        sl = pl.ds(i * 16, 16)
        v = acc[sl]
        acc[sl] = jnp.where(v <= NEG * 0.5, 0.0, v)
        return 0
    lax.fori_loop(0, SPT * D // 16, fix_row, 0)

    pltpu.sync_copy(acc.at[pl.ds(0, SPT * D)],
                    out_hbm.at[pl.ds(pl.multiple_of(lo * D, 128), SPT * D)])


def _drain(x, slst, llst, counts):
    mesh = plsc.VectorSubcoreMesh(core_axis_name="c", subcore_axis_name="s")
    f = pl.kernel(
        _drain_body,
        out_type=jax.ShapeDtypeStruct((SEG * D,), jnp.float32),
        mesh=mesh,
        compiler_params=pltpu.CompilerParams(needs_layout_passes=False),
        scratch_types=[
            pltpu.VMEM((NBUF * GB,), jnp.int32),
            pltpu.VMEM((NBUF * GB,), jnp.int32),
            pltpu.VMEM((16,), jnp.int32),
            pltpu.VMEM((NBUF * GB, D), jnp.float32),
            pltpu.VMEM(((SPT + 1) * D,), jnp.float32),
            pltpu.SemaphoreType.DMA,
            pltpu.SemaphoreType.DMA,
            pltpu.SemaphoreType.DMA,
            pltpu.SemaphoreType.DMA,
            pltpu.SemaphoreType.DMA,
            pltpu.SemaphoreType.DMA,
            pltpu.SemaphoreType.DMA,
            pltpu.SemaphoreType.DMA,
        ],
    )
    return f(x, slst, llst, counts).reshape(SEG, D)


def _combine_body(x_ref, agg_ref, Wroot, broot, Wr0, Wr1, lnw, lnb, out_ref):
    out = (jnp.dot(x_ref[...], Wroot[...], preferred_element_type=jnp.float32)
           + broot[...]
           + jnp.dot(agg_ref[:N, :], Wr0[...], preferred_element_type=jnp.float32)
           + jnp.dot(agg_ref[N:, :], Wr1[...], preferred_element_type=jnp.float32))
    mean = jnp.mean(out)
    std = jnp.sqrt(jnp.mean((out - mean) ** 2))
    out_ref[...] = (out - mean) / (std + 1e-5) * lnw[...] + lnb[...]


def _combine_ln(x, agg, Wroot, broot, Wr0, Wr1, lnw, lnb):
    return pl.pallas_call(
        _combine_body,
        out_shape=jax.ShapeDtypeStruct((N, D), jnp.float32),
    )(x, agg, Wroot, broot, Wr0, Wr1, lnw, lnb)


def _combine_head_body(x_ref, agg_ref, Wroot, broot, Wr0, Wr1, lnw, lnb,
                       Wo1, bo1, Wo2, bo2, out_ref):
    out = (jnp.dot(x_ref[...], Wroot[...], preferred_element_type=jnp.float32)
           + broot[...]
           + jnp.dot(agg_ref[:N, :], Wr0[...], preferred_element_type=jnp.float32)
           + jnp.dot(agg_ref[N:, :], Wr1[...], preferred_element_type=jnp.float32))
    mean = jnp.mean(out)
    std = jnp.sqrt(jnp.mean((out - mean) ** 2))
    out = (out - mean) / (std + 1e-5) * lnw[...] + lnb[...]
    out = _leaky(jnp.dot(out, Wo1[...], preferred_element_type=jnp.float32)
                 + bo1[...])
    logit = jnp.dot(out, Wo2[...], preferred_element_type=jnp.float32) + bo2[...]
    out_ref[...] = jax.nn.sigmoid(logit)


def _combine_head(x, agg, Wroot, broot, Wr0, Wr1, lnw, lnb, Wo1, bo1, Wo2, bo2):
    return pl.pallas_call(
        _combine_head_body,
        out_shape=jax.ShapeDtypeStruct((N, 1), jnp.float32),
    )(x, agg, Wroot, broot, Wr0, Wr1, lnw, lnb, Wo1, bo1, Wo2, bo2)


def kernel(desc_embedding, tweet_embedding, num_feature, cat_feature,
           edge_index, edge_type,
           W_desc, b_desc, W_tweet, b_tweet, W_num, b_num, W_cat, b_cat,
           W_in, b_in, Wr1, Wroot1, broot1, ln1_w, ln1_b,
           Wr2, Wroot2, broot2, ln2_w, ln2_b, W_o1, b_o1, W_o2, b_o2):
    row = lambda b: b.reshape(1, -1).astype(jnp.float32)
    src = edge_index[0].astype(jnp.int32)
    dst = edge_index[1].astype(jnp.int32)
    typ = edge_type.astype(jnp.int32)

    x = _pre(desc_embedding, tweet_embedding, num_feature, cat_feature,
             W_desc, row(b_desc), W_tweet, row(b_tweet),
             W_num, row(b_num), W_cat, row(b_cat), W_in, row(b_in))

    pk = _pack(src, dst, typ)
    slst, llst, counts = _route(pk)

    agg1 = _drain(x, slst, llst, counts)
    x = _combine_ln(x, agg1, Wroot1, row(broot1), Wr1[0], Wr1[1],
                    row(ln1_w), row(ln1_b))

    agg2 = _drain(x, slst, llst, counts)
    out = _combine_head(x, agg2, Wroot2, row(broot2), Wr2[0], Wr2[1],
                        row(ln2_w), row(ln2_b), W_o1, row(b_o1),
                        W_o2, row(b_o2))
    return out.reshape(-1)


# speedup vs baseline: 1.3223x; 1.3223x over previous
import jax
import jax.numpy as jnp
from jax import lax
from jax.experimental import pallas as pl
from jax.experimental.pallas import tpu as pltpu
from jax.experimental.pallas import tpu_sc as plsc

N = 10000
E = 640000
D = 128
NEG = -1e30

NW = 32
SEG = 2 * N
SPT = SEG // NW
CH = 2000
NCHUNK = E // CH
FB = 512
RING = 2 * FB
GB = 64
NBUF = 4
STRIDE = E + 2 * FB
KMASK = 32767


def _leaky(x):
    return jnp.where(x >= 0, x, 0.01 * x)


def _pre_body(desc_ref, tw_ref, nm_ref, ct_ref, Wd, bd, Wt, bt, Wn, bn,
              Wc, bc, Win, bin_, x_ref):
    a = _leaky(jnp.dot(desc_ref[...], Wd[...],
                       preferred_element_type=jnp.float32) + bd[...])
    b = _leaky(jnp.dot(tw_ref[...], Wt[...],
                       preferred_element_type=jnp.float32) + bt[...])
    c = _leaky(jnp.dot(nm_ref[...], Wn[...],
                       preferred_element_type=jnp.float32) + bn[...])
    d = _leaky(jnp.dot(ct_ref[...], Wc[...],
                       preferred_element_type=jnp.float32) + bc[...])
    h = jnp.concatenate([a, b, c, d], axis=1)
    x_ref[...] = _leaky(jnp.dot(h, Win[...],
                                preferred_element_type=jnp.float32) + bin_[...])


def _pre(desc, tw, nm, ct, Wd, bd, Wt, bt, Wn, bn, Wc, bc, Win, bin_):
    BR = 1000
    grid = (N // BR,)
    row_bs = lambda cols: pl.BlockSpec((BR, cols), lambda i: (i, 0))
    full = lambda s: pl.BlockSpec(s, lambda i: (0,) * len(s))
    return pl.pallas_call(
        _pre_body,
        grid=grid,
        in_specs=[row_bs(768), row_bs(768), row_bs(5), row_bs(3),
                  full((768, 32)), full((1, 32)), full((768, 32)), full((1, 32)),
                  full((5, 32)), full((1, 32)), full((3, 32)), full((1, 32)),
                  full((D, D)), full((1, D))],
        out_specs=row_bs(D),
        out_shape=jax.ShapeDtypeStruct((N, D), jnp.float32),
    )(desc, tw, nm, ct, Wd, bd, Wt, bt, Wn, bn, Wc, bc, Win, bin_)


def _pack_body(s_ref, d_ref, t_ref, pk_ref):
    pk_ref[...] = (s_ref[...] << 15) | (d_ref[...] + t_ref[...] * N)


def _pack(src, dst, typ):
    shp = (E // 128, 128)
    return pl.pallas_call(
        _pack_body,
        out_shape=jax.ShapeDtypeStruct(shp, jnp.int32),
    )(src.reshape(shp), dst.reshape(shp), typ.reshape(shp)).reshape(E)


def _route_body(pk_hbm, slist_hbm, llist_hbm, counts_hbm,
                pkv, sring, lring, cntv, c0, c1):
    csems = (c0, c1)
    wid = lax.axis_index("s") * 2 + lax.axis_index("c")
    lo = wid * SPT
    lbase = wid * STRIDE

    def init_ring(i, _):
        sring[pl.ds(i * 16, 16)] = jnp.zeros((16,), jnp.int32)
        lring[pl.ds(i * 16, 16)] = jnp.full((16,), SPT, jnp.int32)
        return 0
    lax.fori_loop(0, RING // 16, init_ring, 0)

    def chunk_slices(ci, p):
        hoff = pl.multiple_of(ci * CH, 16)
        return pk_hbm.at[pl.ds(hoff, CH)], pkv.at[pl.ds(p * CH, CH)]

    def issue(ci, p):
        s, v = chunk_slices(ci, p)
        pltpu.async_copy(s, v, csems[p])

    def wait_chunk(p):
        s, v = chunk_slices(0, p)
        pltpu.make_async_copy(s, v, csems[p]).wait()

    def flush(block, sel):
        boff = pl.multiple_of(sel * FB, FB)
        hoff = pl.multiple_of(lbase + block * FB, FB)
        pltpu.sync_copy(sring.at[pl.ds(boff, FB)], slist_hbm.at[pl.ds(hoff, FB)])
        pltpu.sync_copy(lring.at[pl.ds(boff, FB)], llist_hbm.at[pl.ds(hoff, FB)])

    def scan_one(off, c):
        cnt, flushed = c
        pkx = pkv[pl.ds(off, 16)]
        rel = (pkx & KMASK) - lo
        msk = plsc.bitcast(rel, jnp.uint32) < jnp.uint32(SPT)
        csum = jnp.cumsum(jnp.where(msk, 1, 0).astype(jnp.int32))
        idx = (cnt + csum - 1) & (RING - 1)
        plsc.store_scatter(sring, [idx], pkx >> 15, mask=msk)
        plsc.store_scatter(lring, [idx], rel, mask=msk)
        return cnt + plsc.all_reduce_population_count(msk)[0], flushed

    def scan_slot(p, carry):
        def scan_v(v, c):
            base = p * CH + v * 16
            cnt, flushed = scan_one(base, c)

            @pl.when(cnt - flushed >= FB)
            def _():
                flush(flushed // FB, (flushed // FB) & 1)
            flushed = jnp.where(cnt - flushed >= FB, flushed + FB, flushed)
            return (cnt, flushed)
        return lax.fori_loop(0, CH // 16, scan_v, carry)

    issue(0, 0)
    issue(1, 1)

    def super_step(c2, carry):
        for p in range(2):
            wait_chunk(p)
            carry = scan_slot(p, carry)
            issue(c2 * 2 + p + 2, p)
        return carry

    carry = lax.fori_loop(0, NCHUNK // 2 - 1, super_step,
                          (jnp.int32(0), jnp.int32(0)))
    for p in range(2):
        wait_chunk(p)
        carry = scan_slot(p, carry)
    cnt, flushed = carry

    pad_idx = (cnt + lax.iota(jnp.int32, 16)) & (RING - 1)
    plsc.store_scatter(sring, [pad_idx], jnp.zeros((16,), jnp.int32))
    plsc.store_scatter(lring, [pad_idx], jnp.full((16,), SPT, jnp.int32))
    nblk = (cnt + 16 + FB - 1) // FB
    for extra in range(2):
        bidx = flushed // FB + extra

        @pl.when(bidx < nblk)
        def _():
            flush(bidx, bidx & 1)

    cntv[pl.ds(0, 16)] = jnp.broadcast_to(nblk * FB, (16,)).astype(jnp.int32)
    pltpu.sync_copy(cntv, counts_hbm.at[pl.ds(pl.multiple_of(wid * 16, 16), 16)])


def _route(pk):
    mesh = plsc.VectorSubcoreMesh(core_axis_name="c", subcore_axis_name="s")
    f = pl.kernel(
        _route_body,
        out_type=(jax.ShapeDtypeStruct((NW * STRIDE,), jnp.int32),
                  jax.ShapeDtypeStruct((NW * STRIDE,), jnp.int32),
                  jax.ShapeDtypeStruct((NW * 16,), jnp.int32)),
        mesh=mesh,
        compiler_params=pltpu.CompilerParams(needs_layout_passes=False),
        scratch_types=[
            pltpu.VMEM((2 * CH,), jnp.int32),
            pltpu.VMEM((RING,), jnp.int32),
            pltpu.VMEM((RING,), jnp.int32),
            pltpu.VMEM((16,), jnp.int32),
            pltpu.SemaphoreType.DMA,
            pltpu.SemaphoreType.DMA,
        ],
    )
    return f(pk)


def _drain_body(x_hbm, slist_hbm, llist_hbm, counts_hbm, out_hbm,
                idxb, locb, cntv, rows, acc, s0, s1, s2, s3):
    sems = (s0, s1, s2, s3)
    SB = NBUF * GB
    wid = lax.axis_index("s") * 2 + lax.axis_index("c")
    lo = wid * SPT
    lbase = wid * STRIDE

    def init_row(i, _):
        acc[pl.ds(i * 16, 16)] = jnp.full((16,), NEG, jnp.float32)
        return 0
    lax.fori_loop(0, (SPT + 1) * D // 16, init_row, 0)

    pltpu.sync_copy(counts_hbm.at[pl.ds(pl.multiple_of(wid * 16, 16), 16)], cntv)
    m = cntv[pl.ds(0, 16)][0]
    nb = m // GB
    SS = nb // NBUF

    def load_sb(sb, q):
        hoff = pl.multiple_of(lbase + sb * SB, SB)
        pltpu.sync_copy(slist_hbm.at[pl.ds(hoff, SB)],
                        idxb.at[pl.ds(q * SB, SB)])
        pltpu.sync_copy(llist_hbm.at[pl.ds(hoff, SB)],
                        locb.at[pl.ds(q * SB, SB)])

    def issue(q, p):
        pltpu.async_copy(x_hbm.at[idxb.at[pl.ds(q * SB + p * GB, GB)]],
                         rows.at[pl.ds(p * GB, GB)], sems[p])

    def wait_slot(q, p):
        pltpu.make_async_copy(x_hbm.at[idxb.at[pl.ds(q * SB + p * GB, GB)]],
                              rows.at[pl.ds(p * GB, GB)], sems[p]).wait()

    def rmw_slot(q, p):
        def group_step(g, _):
            locs = locb[pl.ds(q * SB + p * GB + g * 16, 16)]
            for k16 in range(16):
                rbase = pl.multiple_of(locs[k16] * D, D)
                r = p * GB + g * 16 + k16
                for k in range(8):
                    sl = pl.ds(rbase + k * 16, 16)
                    acc[sl] = jnp.maximum(acc[sl], rows[r, pl.ds(k * 16, 16)])
            return 0
        lax.fori_loop(0, GB // 16, group_step, 0)

    def step(q, load_next):
        for p in range(NBUF):
            wait_slot(q, p)
            rmw_slot(q, p)
            issue(1 - q, p)
        if load_next is not None:
            load_sb(load_next, q)

    load_sb(jnp.int32(0), 0)
    for p in range(NBUF):
        issue(0, p)
    load_sb(jnp.int32(1), 1)

    def pair_step(ss2, _):
        step(0, 2 * ss2 + 2)
        step(1, 2 * ss2 + 3)
        return 0
    lax.fori_loop(0, (SS - 2) // 2, pair_step, 0)

    for p in range(NBUF):
        wait_slot(0, p)
        rmw_slot(0, p)
        issue(1, p)
    for p in range(NBUF):
        wait_slot(1, p)
        rmw_slot(1, p)

    def fix_row(i, _):
        sl = pl.ds(i * 16, 16)
        v = acc[sl]
        acc[sl] = jnp.where(v <= NEG * 0.5, 0.0, v)
        return 0
    lax.fori_loop(0, SPT * D // 16, fix_row, 0)

    pltpu.sync_copy(acc.at[pl.ds(0, SPT * D)],
                    out_hbm.at[pl.ds(pl.multiple_of(lo * D, 128), SPT * D)])


def _drain(x, slst, llst, counts):
    mesh = plsc.VectorSubcoreMesh(core_axis_name="c", subcore_axis_name="s")
    f = pl.kernel(
        _drain_body,
        out_type=jax.ShapeDtypeStruct((SEG * D,), jnp.float32),
        mesh=mesh,
        compiler_params=pltpu.CompilerParams(needs_layout_passes=False),
        scratch_types=[
            pltpu.VMEM((2 * NBUF * GB,), jnp.int32),
            pltpu.VMEM((2 * NBUF * GB,), jnp.int32),
            pltpu.VMEM((16,), jnp.int32),
            pltpu.VMEM((NBUF * GB, D), jnp.float32),
            pltpu.VMEM(((SPT + 1) * D,), jnp.float32),
            pltpu.SemaphoreType.DMA,
            pltpu.SemaphoreType.DMA,
            pltpu.SemaphoreType.DMA,
            pltpu.SemaphoreType.DMA,
        ],
    )
    return f(x, slst, llst, counts).reshape(SEG, D)


def _combine_body(x_ref, agg_ref, Wroot, broot, Wr0, Wr1, lnw, lnb, out_ref):
    out = (jnp.dot(x_ref[...], Wroot[...], preferred_element_type=jnp.float32)
           + broot[...]
           + jnp.dot(agg_ref[:N, :], Wr0[...], preferred_element_type=jnp.float32)
           + jnp.dot(agg_ref[N:, :], Wr1[...], preferred_element_type=jnp.float32))
    mean = jnp.mean(out)
    std = jnp.sqrt(jnp.mean((out - mean) ** 2))
    out_ref[...] = (out - mean) / (std + 1e-5) * lnw[...] + lnb[...]


def _combine_ln(x, agg, Wroot, broot, Wr0, Wr1, lnw, lnb):
    return pl.pallas_call(
        _combine_body,
        out_shape=jax.ShapeDtypeStruct((N, D), jnp.float32),
    )(x, agg, Wroot, broot, Wr0, Wr1, lnw, lnb)


def _combine_head_body(x_ref, agg_ref, Wroot, broot, Wr0, Wr1, lnw, lnb,
                       Wo1, bo1, Wo2, bo2, out_ref):
    out = (jnp.dot(x_ref[...], Wroot[...], preferred_element_type=jnp.float32)
           + broot[...]
           + jnp.dot(agg_ref[:N, :], Wr0[...], preferred_element_type=jnp.float32)
           + jnp.dot(agg_ref[N:, :], Wr1[...], preferred_element_type=jnp.float32))
    mean = jnp.mean(out)
    std = jnp.sqrt(jnp.mean((out - mean) ** 2))
    out = (out - mean) / (std + 1e-5) * lnw[...] + lnb[...]
    out = _leaky(jnp.dot(out, Wo1[...], preferred_element_type=jnp.float32)
                 + bo1[...])
    logit = jnp.dot(out, Wo2[...], preferred_element_type=jnp.float32) + bo2[...]
    out_ref[...] = jax.nn.sigmoid(logit)


def _combine_head(x, agg, Wroot, broot, Wr0, Wr1, lnw, lnb, Wo1, bo1, Wo2, bo2):
    return pl.pallas_call(
        _combine_head_body,
        out_shape=jax.ShapeDtypeStruct((N, 1), jnp.float32),
    )(x, agg, Wroot, broot, Wr0, Wr1, lnw, lnb, Wo1, bo1, Wo2, bo2)


def kernel(desc_embedding, tweet_embedding, num_feature, cat_feature,
           edge_index, edge_type,
           W_desc, b_desc, W_tweet, b_tweet, W_num, b_num, W_cat, b_cat,
           W_in, b_in, Wr1, Wroot1, broot1, ln1_w, ln1_b,
           Wr2, Wroot2, broot2, ln2_w, ln2_b, W_o1, b_o1, W_o2, b_o2):
    row = lambda b: b.reshape(1, -1).astype(jnp.float32)
    src = edge_index[0].astype(jnp.int32)
    dst = edge_index[1].astype(jnp.int32)
    typ = edge_type.astype(jnp.int32)

    x = _pre(desc_embedding, tweet_embedding, num_feature, cat_feature,
             W_desc, row(b_desc), W_tweet, row(b_tweet),
             W_num, row(b_num), W_cat, row(b_cat), W_in, row(b_in))

    pk = _pack(src, dst, typ)
    slst, llst, counts = _route(pk)

    agg1 = _drain(x, slst, llst, counts)
    x = _combine_ln(x, agg1, Wroot1, row(broot1), Wr1[0], Wr1[1],
                    row(ln1_w), row(ln1_b))

    agg2 = _drain(x, slst, llst, counts)
    out = _combine_head(x, agg2, Wroot2, row(broot2), Wr2[0], Wr2[1],
                        row(ln2_w), row(ln2_b), W_o1, row(b_o1),
                        W_o2, row(b_o2))
    return out.reshape(-1)
